# baseline (speedup 1.0000x reference)
import jax
import jax.numpy as jnp
from jax import lax
from jax.experimental import pallas as pl
from jax.experimental.pallas import tpu as pltpu
from jax.experimental.pallas import tpu_sc as plsc

B = 16384
L = 50
E = 32

NC = 2
NS = 16
NW = NC * NS
PAIRS = B * L
PPW = PAIRS // NW
BPW = B // NW
CHUNK = 128
NCHUNK = PPW // CHUNK
SUPER = 4
SPAIRS = SUPER * CHUNK
NSUPER = NCHUNK // SUPER


def _fm_body(item2d, ur, i_table, ub, ib, out,
             idx_v, u_rows, ub_v, i_rows, ib_v, out_v, semA, semB):
    c = lax.axis_index("c")
    s = lax.axis_index("s")
    wid = s * NC + c
    pair0 = wid * PPW
    b0 = wid * BPW

    pltpu.sync_copy(item2d.at[pl.ds(wid * NCHUNK, NCHUNK)], idx_v)
    pltpu.sync_copy(ur.at[pl.ds(b0, BPW)], u_rows)
    pltpu.sync_copy(ub.at[pl.ds(b0, BPW)], ub_v)

    iota = lax.iota(jnp.int32, 16)

    def issue(sc, buf, sem):
        for j in range(SUPER):
            row = sc * SUPER + j
            pltpu.async_copy(i_table.at[idx_v.at[row]],
                             i_rows.at[buf].at[pl.ds(j * CHUNK, CHUNK)], sem)
            pltpu.async_copy(ib.at[idx_v.at[row]],
                             ib_v.at[buf].at[pl.ds(j * CHUNK, CHUNK)], sem)

    def drain(buf, sem):
        for j in range(SUPER):
            pltpu.make_async_copy(
                i_table.at[pl.ds(0, CHUNK)],
                i_rows.at[buf].at[pl.ds(j * CHUNK, CHUNK)], sem).wait()
            pltpu.make_async_copy(
                ib.at[pl.ds(0, CHUNK)],
                ib_v.at[buf].at[pl.ds(j * CHUNK, CHUNK)], sem).wait()

    def compute(sc, buf):
        rows = i_rows.at[buf]
        biases = ib_v.at[buf]

        def group_body(g, carry):
            lg = g * 16 + iota
            p = pair0 + sc * SPAIRS + g * 16 + iota
            b_loc = lax.div(p, L) - b0
            acc = (plsc.load_gather(ub_v, [b_loc])
                   + plsc.load_gather(biases, [lg]))
            for e in range(E):
                ecol = jnp.full((16,), e, jnp.int32)
                acc = acc + (plsc.load_gather(rows, [lg, ecol])
                             * plsc.load_gather(u_rows, [b_loc, ecol]))
            plsc.store_scatter(out_v, [sc * SPAIRS + g * 16 + iota], acc)
            return carry

        lax.fori_loop(0, SPAIRS // 16, group_body, 0)

    issue(0, 0, semA)

    def pair_body(k, carry):
        sc0 = 2 * k
        sc1 = 2 * k + 1
        issue(sc1, 1, semB)
        drain(0, semA)
        compute(sc0, 0)

        @pl.when(k < NSUPER // 2 - 1)
        def _():
            issue(sc0 + 2, 0, semA)

        drain(1, semB)
        compute(sc1, 1)
        return carry

    lax.fori_loop(0, NSUPER // 2, pair_body, 0)
    pltpu.sync_copy(out_v, out.at[pl.ds(pair0, PPW)])


@jax.jit
def _fm(item2d, ur, i_table, ub, ib):
    mesh = plsc.VectorSubcoreMesh(core_axis_name="c", subcore_axis_name="s",
                                  num_cores=NC, num_subcores=NS)
    return pl.kernel(
        _fm_body,
        out_type=jax.ShapeDtypeStruct((PAIRS,), jnp.float32),
        mesh=mesh,
        compiler_params=pltpu.CompilerParams(needs_layout_passes=False,
                                             use_tc_tiling_on_sc=False),
        scratch_types=[
            pltpu.VMEM((NCHUNK, CHUNK), jnp.int32),
            pltpu.VMEM((BPW, E), jnp.float32),
            pltpu.VMEM((BPW,), jnp.float32),
            pltpu.VMEM((2, SPAIRS, E), jnp.float32),
            pltpu.VMEM((2, SPAIRS), jnp.float32),
            pltpu.VMEM((PPW,), jnp.float32),
            pltpu.SemaphoreType.DMA,
            pltpu.SemaphoreType.DMA,
        ],
    )(item2d, ur, i_table, ub, ib)


def kernel(user, item, u_table, i_table, u_bias, i_bias):
    uids = user.astype(jnp.int32)[:, 0]
    item2d = item.astype(jnp.int32).reshape(PAIRS // CHUNK, CHUNK)
    ur = jnp.take(u_table, uids, axis=0)
    ub = jnp.take(u_bias[:, 0], uids)
    ibc = jnp.take(i_bias[:, 0], jnp.arange(i_bias.shape[0], dtype=jnp.int32))
    out = _fm(item2d, ur, i_table, ub, ibc)
    return out.reshape(B, L)


# speedup vs baseline: 1.0275x; 1.0275x over previous
import jax
import jax.numpy as jnp
from jax import lax
from jax.experimental import pallas as pl
from jax.experimental.pallas import tpu as pltpu
from jax.experimental.pallas import tpu_sc as plsc

B = 16384
L = 50
E = 32

NC = 2
NS = 16
NW = NC * NS
PAIRS = B * L
PPW = PAIRS // NW
BPW = B // NW
CHUNK = 128
NCHUNK = PPW // CHUNK
SUPER = 4
SPAIRS = SUPER * CHUNK
NSUPER = NCHUNK // SUPER

_SC_PARAMS = pltpu.CompilerParams(needs_layout_passes=False,
                                  use_tc_tiling_on_sc=False)


def _mesh():
    return plsc.VectorSubcoreMesh(core_axis_name="c", subcore_axis_name="s",
                                  num_cores=NC, num_subcores=NS)


def _worker():
    return lax.axis_index("s") * NC + lax.axis_index("c")


def _pipeline(issue, drain, compute):
    issue(0, 0, 0)

    def pair_body(k, carry):
        sc0 = 2 * k
        issue(sc0 + 1, 1, 1)
        drain(0, 0)
        compute(sc0, 0)

        @pl.when(k < NSUPER // 2 - 1)
        def _():
            issue(sc0 + 2, 0, 0)

        drain(1, 1)
        compute(sc0 + 1, 1)
        return carry

    lax.fori_loop(0, NSUPER // 2, pair_body, 0)


def _fm_body(item2d, ur, i_table, ub, out,
             idx_v, u_rows, ub_v, i_rows, out_v, semA, semB):
    wid = _worker()
    pair0 = wid * PPW
    b0 = wid * BPW
    sems = (semA, semB)

    pltpu.sync_copy(item2d.at[pl.ds(wid * NCHUNK, NCHUNK)], idx_v)
    pltpu.sync_copy(ur.at[pl.ds(b0, BPW)], u_rows)
    pltpu.sync_copy(ub.at[pl.ds(b0, BPW)], ub_v)

    iota = lax.iota(jnp.int32, 16)

    def issue(sc, buf, sem):
        for j in range(SUPER):
            pltpu.async_copy(i_table.at[idx_v.at[sc * SUPER + j]],
                             i_rows.at[buf].at[pl.ds(j * CHUNK, CHUNK)],
                             sems[sem])

    def drain(buf, sem):
        for j in range(SUPER):
            pltpu.make_async_copy(
                i_table.at[pl.ds(0, CHUNK)],
                i_rows.at[buf].at[pl.ds(j * CHUNK, CHUNK)], sems[sem]).wait()

    def compute(sc, buf):
        rows = i_rows.at[buf]

        def group_body(g, carry):
            lg = g * 16 + iota
            p = pair0 + sc * SPAIRS + g * 16 + iota
            b_loc = lax.div(p, L) - b0
            acc = plsc.load_gather(ub_v, [b_loc])
            for e in range(E):
                ecol = jnp.full((16,), e, jnp.int32)
                acc = acc + (plsc.load_gather(rows, [lg, ecol])
                             * plsc.load_gather(u_rows, [b_loc, ecol]))
            plsc.store_scatter(out_v, [sc * SPAIRS + g * 16 + iota], acc)
            return carry

        lax.fori_loop(0, SPAIRS // 16, group_body, 0)

    _pipeline(issue, drain, compute)
    pltpu.sync_copy(out_v, out.at[pl.ds(pair0, PPW)])


def _bias_body(item2d, ib, part, out,
               idx_v, ib_v, part_v, out_v, semA, semB):
    wid = _worker()
    pair0 = wid * PPW
    sems = (semA, semB)

    pltpu.sync_copy(item2d.at[pl.ds(wid * NCHUNK, NCHUNK)], idx_v)

    iota = lax.iota(jnp.int32, 16)

    def issue(sc, buf, sem):
        for j in range(SUPER):
            pltpu.async_copy(ib.at[idx_v.at[sc * SUPER + j]],
                             ib_v.at[buf].at[pl.ds(j * CHUNK, CHUNK)],
                             sems[sem])
        pltpu.async_copy(part.at[pl.ds(pair0 + sc * SPAIRS, SPAIRS)],
                         part_v.at[buf], sems[sem])

    def drain(buf, sem):
        for j in range(SUPER):
            pltpu.make_async_copy(
                ib.at[pl.ds(0, CHUNK)],
                ib_v.at[buf].at[pl.ds(j * CHUNK, CHUNK)], sems[sem]).wait()
        pltpu.make_async_copy(part.at[pl.ds(0, SPAIRS)],
                              part_v.at[buf], sems[sem]).wait()

    def compute(sc, buf):
        biases = ib_v.at[buf]
        parts = part_v.at[buf]

        def group_body(g, carry):
            lg = g * 16 + iota
            acc = (plsc.load_gather(parts, [lg])
                   + plsc.load_gather(biases, [lg]))
            plsc.store_scatter(out_v, [sc * SPAIRS + g * 16 + iota], acc)
            return carry

        lax.fori_loop(0, SPAIRS // 16, group_body, 0)

    _pipeline(issue, drain, compute)
    pltpu.sync_copy(out_v, out.at[pl.ds(pair0, PPW)])


@jax.jit
def _fm_full(item2d, ur, i_table, ub, i_bias2d):
    part = pl.kernel(
        _fm_body,
        out_type=jax.ShapeDtypeStruct((PAIRS,), jnp.float32),
        mesh=_mesh(),
        compiler_params=_SC_PARAMS,
        scratch_types=[
            pltpu.VMEM((NCHUNK, CHUNK), jnp.int32),
            pltpu.VMEM((BPW, E), jnp.float32),
            pltpu.VMEM((BPW,), jnp.float32),
            pltpu.VMEM((2, SPAIRS, E), jnp.float32),
            pltpu.VMEM((PPW,), jnp.float32),
            pltpu.SemaphoreType.DMA,
            pltpu.SemaphoreType.DMA,
        ],
    )(item2d, ur, i_table, ub)
    ibc = i_bias2d.reshape(-1)
    return pl.kernel(
        _bias_body,
        out_type=jax.ShapeDtypeStruct((PAIRS,), jnp.float32),
        mesh=_mesh(),
        compiler_params=_SC_PARAMS,
        scratch_types=[
            pltpu.VMEM((NCHUNK, CHUNK), jnp.int32),
            pltpu.VMEM((2, SPAIRS), jnp.float32),
            pltpu.VMEM((2, SPAIRS), jnp.float32),
            pltpu.VMEM((PPW,), jnp.float32),
            pltpu.SemaphoreType.DMA,
            pltpu.SemaphoreType.DMA,
        ],
    )(item2d, ibc, part)


def kernel(user, item, u_table, i_table, u_bias, i_bias):
    uids = user.astype(jnp.int32)[:, 0]
    item2d = item.astype(jnp.int32).reshape(PAIRS // CHUNK, CHUNK)
    ur = jnp.take(u_table, uids, axis=0)
    ub = jnp.take(u_bias[:, 0], uids)
    out = _fm_full(item2d, ur, i_table, ub, i_bias)
    return out.reshape(B, L)


# speedup vs baseline: 1.0695x; 1.0409x over previous
import jax
import jax.numpy as jnp
from jax import lax
from jax.experimental import pallas as pl
from jax.experimental.pallas import tpu as pltpu
from jax.experimental.pallas import tpu_sc as plsc

B = 16384
L = 50
E = 32

NC = 2
NS = 16
NW = NC * NS
PAIRS = B * L
PPW = PAIRS // NW
BPW = B // NW
SPAIRS = 512
NCHUNK = PPW // SPAIRS
NSUPER = NCHUNK


def _fm_body(item2d, ur, i_table, ub, ib, out,
             idx_v, u_rows, ub_v, i_rows, ib_v, out_v, semA, semB):
    c = lax.axis_index("c")
    s = lax.axis_index("s")
    wid = s * NC + c
    pair0 = wid * PPW
    b0 = wid * BPW
    sems = (semA, semB)

    pltpu.sync_copy(item2d.at[pl.ds(wid * NCHUNK, NCHUNK)], idx_v)
    pltpu.sync_copy(ur.at[pl.ds(b0, BPW)], u_rows)
    pltpu.sync_copy(ub.at[pl.ds(b0, BPW)], ub_v)

    iota = lax.iota(jnp.int32, 16)

    def issue(sc, buf, sem):
        idx = idx_v.at[sc]
        pltpu.async_copy(i_table.at[idx], i_rows.at[buf], sems[sem])
        pltpu.async_copy(ib.at[idx], ib_v.at[buf], sems[sem])

    def drain(buf, sem):
        idx0 = idx_v.at[0]
        pltpu.make_async_copy(i_table.at[idx0],
                              i_rows.at[buf], sems[sem]).wait()
        pltpu.make_async_copy(ib.at[idx0],
                              ib_v.at[buf], sems[sem]).wait()

    def compute(sc, buf):
        rows = i_rows.at[buf]
        biases = ib_v.at[buf]

        def group_body(g, carry):
            lg = g * 16 + iota
            p = pair0 + sc * SPAIRS + g * 16 + iota
            b_loc = lax.div(p, L) - b0
            acc = (plsc.load_gather(ub_v, [b_loc])
                   + plsc.load_gather(biases, [lg]))
            for e in range(E):
                ecol = jnp.full((16,), e, jnp.int32)
                acc = acc + (plsc.load_gather(rows, [lg, ecol])
                             * plsc.load_gather(u_rows, [b_loc, ecol]))
            plsc.store_scatter(out_v, [sc * SPAIRS + g * 16 + iota], acc)
            return carry

        lax.fori_loop(0, SPAIRS // 16, group_body, 0)

    issue(0, 0, 0)

    def pair_body(k, carry):
        sc0 = 2 * k
        issue(sc0 + 1, 1, 1)
        drain(0, 0)
        compute(sc0, 0)

        @pl.when(k < NSUPER // 2 - 1)
        def _():
            issue(sc0 + 2, 0, 0)

        drain(1, 1)
        compute(sc0 + 1, 1)
        return carry

    lax.fori_loop(0, NSUPER // 2, pair_body, 0)
    pltpu.sync_copy(out_v, out.at[pl.ds(pair0, PPW)])


@jax.jit
def _fm(item2d, ur, i_table, ub, ib):
    mesh = plsc.VectorSubcoreMesh(core_axis_name="c", subcore_axis_name="s",
                                  num_cores=NC, num_subcores=NS)
    return pl.kernel(
        _fm_body,
        out_type=jax.ShapeDtypeStruct((PAIRS,), jnp.float32),
        mesh=mesh,
        compiler_params=pltpu.CompilerParams(needs_layout_passes=False,
                                             use_tc_tiling_on_sc=False),
        scratch_types=[
            pltpu.VMEM((NCHUNK, SPAIRS), jnp.int32),
            pltpu.VMEM((BPW, E), jnp.float32),
            pltpu.VMEM((BPW,), jnp.float32),
            pltpu.VMEM((2, SPAIRS, E), jnp.float32),
            pltpu.VMEM((2, SPAIRS), jnp.float32),
            pltpu.VMEM((PPW,), jnp.float32),
            pltpu.SemaphoreType.DMA,
            pltpu.SemaphoreType.DMA,
        ],
    )(item2d, ur, i_table, ub, ib)


def kernel(user, item, u_table, i_table, u_bias, i_bias):
    uids = user.astype(jnp.int32)[:, 0]
    item2d = item.astype(jnp.int32).reshape(PAIRS // SPAIRS, SPAIRS)
    ur = jnp.take(u_table, uids, axis=0)
    ub = jnp.take(u_bias[:, 0], uids)
    out = _fm(item2d, ur, i_table, ub, i_bias.reshape(-1))
    return out.reshape(B, L)
